# baseline (device time: 51644 ns/iter reference)
import os

import jax
import jax.numpy as jnp
from jax import lax
from jax.experimental import pallas as pl
from jax.experimental.pallas import tpu as pltpu

_NO_COMM = os.environ.get("KERNEL_NO_COMM") == "1"

N_DEV = 8
B = 2
SQ = 512
SKV = 512
H_LOC = 8
DH = 64
D_MODEL = 768
CHUNK = SQ // N_DEV


def kernel(x, Wq, K_ext, V_ext, Wo):
    idx = lax.axis_index("i")

    Kf = K_ext.reshape(B, SKV, 64 * DH)
    Vf = V_ext.reshape(B, SKV, 64 * DH)
    K_sh = lax.dynamic_slice_in_dim(
        Kf, idx * (H_LOC * DH), H_LOC * DH, axis=2).astype(jnp.bfloat16)
    V_sh = lax.dynamic_slice_in_dim(
        Vf, idx * (H_LOC * DH), H_LOC * DH, axis=2).astype(jnp.bfloat16)

    def body(x_ref, wq_ref, k_ref, v_ref, wo_ref, out_ref,
             acc_ref, agbuf,
             sbuf0, rbuf0, sA1, rA1, sB1, rB1, sA2, rA2, sB2, rB2,
             rs_send, rs_recv, rs2_send, rs2_recv, ag_send, ag_recv):
        my = lax.axis_index("i")
        p_z = jnp.bitwise_xor(my, 4)
        p_y = jnp.bitwise_xor(my, 3)
        p_x = jnp.bitwise_xor(my, 1)

        b2 = my // 4
        b1 = (my // 2) % 2
        b0 = my % 2
        u2 = b2
        u1 = b1
        u0 = jnp.bitwise_xor(b1, b0)
        u = 4 * u2 + 2 * u1 + u0

        barrier_sem = pltpu.get_barrier_semaphore()
        for nbr in (p_z, p_y, p_x):
            pl.semaphore_signal(
                barrier_sem, inc=1,
                device_id=(nbr,), device_id_type=pl.DeviceIdType.MESH,
            )
        pl.semaphore_wait(barrier_sem, 3)

        HALF = SQ // 2
        wq = (wq_ref[...] * 0.125).astype(jnp.bfloat16)
        wo = wo_ref[...].astype(jnp.bfloat16)
        kbf = [k_ref[b] for b in range(B)]
        vbf = [v_ref[b] for b in range(B)]

        KB_LISTS = {
            0: [0, 3, 6], 1: [0, 1, 2, 5], 2: [0, 1, 2, 4, 7],
            3: [0, 3, 6], 4: [0, 2, 4, 5], 5: [0, 1, 4, 5, 7],
            6: [0, 3, 6], 7: [0, 2, 5, 7],
        }

        def compute_half(r0, b):
            xb = x_ref[b, r0:r0 + HALF, :].astype(jnp.bfloat16)
            q = jnp.dot(xb, wq, preferred_element_type=jnp.float32)
            qbf = q.astype(jnp.bfloat16)
            ctx_rows = []
            for qb_ in range(r0 // 64, r0 // 64 + 4):
                kbs = KB_LISTS[qb_]
                ksel = jnp.concatenate(
                    [kbf[b][kb * 64:(kb + 1) * 64, :] for kb in kbs], axis=0)
                vsel = jnp.concatenate(
                    [vbf[b][kb * 64:(kb + 1) * 64, :] for kb in kbs], axis=0)
                qrows = qbf[qb_ * 64 - r0:qb_ * 64 - r0 + 64, :]
                hparts = []
                for h in range(H_LOC):
                    sl = slice(h * DH, (h + 1) * DH)
                    s = lax.dot_general(
                        qrows[:, sl], ksel[:, sl], (((1,), (1,)), ((), ())),
                        preferred_element_type=jnp.float32,
                    )
                    w = jnp.exp(s)
                    rcp = 1.0 / jnp.sum(w, axis=1, keepdims=True)
                    hparts.append(
                        jnp.dot(w.astype(jnp.bfloat16), vsel[:, sl],
                                preferred_element_type=jnp.float32) * rcp
                    )
                ctx_rows.append(jnp.concatenate(hparts, axis=1))
            ctx = jnp.concatenate(ctx_rows, axis=0).astype(jnp.bfloat16)
            part = jnp.dot(ctx, wo, preferred_element_type=jnp.float32)
            acc_ref[r0 // CHUNK:r0 // CHUNK + 4, b] = part.reshape(
                HALF // CHUNK, CHUNK, D_MODEL
            )

        def compute_half_for(which, b):
            @pl.when(u2 == 0)
            def _():
                compute_half(HALF if which == "send" else 0, b)

            @pl.when(u2 == 1)
            def _():
                compute_half(0 if which == "send" else HALF, b)

        if _NO_COMM:
            for b in range(B):
                compute_half_for("send", b)
                compute_half_for("keep", b)
            for c in range(N_DEV):
                out_ref[:, c * CHUNK:(c + 1) * CHUNK, :] = (
                    acc_ref[c].astype(jnp.bfloat16))
            return

        rdma0 = []
        for b in range(B):
            compute_half_for("send", b)
            sbuf0[:, b] = acc_ref[pl.ds(4 * (1 - u2), 4), b].astype(
                jnp.bfloat16)
            r = pltpu.make_async_remote_copy(
                src_ref=sbuf0.at[:, b],
                dst_ref=rbuf0.at[:, b],
                send_sem=rs_send.at[b],
                recv_sem=rs_recv.at[b],
                device_id=(p_z,),
                device_id_type=pl.DeviceIdType.MESH,
            )
            r.start()
            rdma0.append(r)
        for b in range(B):
            compute_half_for("keep", b)
        for r in rdma0:
            r.wait()

        base = 4 * u2
        CA = D_MODEL // 2

        def start_rdma(src, dst, ssem, rsem, partner):
            rdma = pltpu.make_async_remote_copy(
                src_ref=src, dst_ref=dst, send_sem=ssem, recv_sem=rsem,
                device_id=(partner,), device_id_type=pl.DeviceIdType.MESH,
            )
            rdma.start()
            return rdma

        def add_r0(lo, width, c0=0, c1=D_MODEL):
            sl = pl.ds(lo, width)
            acc_ref[sl, :, :, c0:c1] = (
                acc_ref[sl, :, :, c0:c1]
                + rbuf0[pl.ds(lo - 4 * u2, width), :, :, c0:c1
                        ].astype(jnp.float32)
            )

        add_r0(base + 2 * (1 - u1), 2)
        add_r0(base + 2 * u1 + (1 - u0), 1, c0=CA)
        sA1[...] = acc_ref[pl.ds(base + 2 * (1 - u1), 2), :, :, :CA].astype(
            jnp.bfloat16)
        r_a1 = start_rdma(sA1, rA1, rs2_send.at[0], rs2_recv.at[0], p_y)
        r_b1 = []
        for k in range(2):
            sB1[k] = acc_ref[base + 2 * k + (1 - u0), :, :, CA:].astype(
                jnp.bfloat16)
            r_b1.append(start_rdma(
                sB1.at[k], rB1.at[k],
                rs2_send.at[1 + k], rs2_recv.at[1 + k], p_x))
        add_r0(base + 2 * u1 + u0, 1)
        add_r0(base + 2 * u1 + (1 - u0), 1, c1=CA)
        r_a1.wait()
        for r in r_b1:
            r.wait()

        acc_ref[pl.ds(base + 2 * u1 + (1 - u0), 1), :, :, :CA] = (
            acc_ref[pl.ds(base + 2 * u1 + (1 - u0), 1), :, :, :CA]
            + rA1[pl.ds(1 - u0, 1)].astype(jnp.float32)
        )
        acc_ref[pl.ds(base + 2 * (1 - u1) + u0, 1), :, :, CA:] = (
            acc_ref[pl.ds(base + 2 * (1 - u1) + u0, 1), :, :, CA:]
            + rB1[pl.ds(1 - u1, 1)].astype(jnp.float32)
        )

        sA2[...] = acc_ref[pl.ds(base + 2 * u1 + (1 - u0), 1), :, :, :CA
                           ].astype(jnp.bfloat16)
        r_a2 = start_rdma(sA2, rA2, rs2_send.at[3], rs2_recv.at[3], p_x)
        sB2[...] = acc_ref[pl.ds(base + 2 * (1 - u1) + u0, 1), :, :, CA:
                           ].astype(jnp.bfloat16)
        r_b2 = start_rdma(sB2, rB2, rs2_send.at[4], rs2_recv.at[4], p_y)
        acc_ref[pl.ds(u, 1), :, :, :CA] = (
            acc_ref[pl.ds(u, 1), :, :, :CA]
            + rA1[pl.ds(u0, 1)].astype(jnp.float32)
        )
        acc_ref[pl.ds(u, 1), :, :, CA:] = (
            acc_ref[pl.ds(u, 1), :, :, CA:]
            + rB1[pl.ds(u1, 1)].astype(jnp.float32)
        )
        r_a2.wait()
        r_b2.wait()
        acc_ref[pl.ds(u, 1), :, :, :CA] = (
            acc_ref[pl.ds(u, 1), :, :, :CA] + rA2[...].astype(jnp.float32)
        )
        acc_ref[pl.ds(u, 1), :, :, CA:] = (
            acc_ref[pl.ds(u, 1), :, :, CA:] + rB2[...].astype(jnp.float32)
        )

        PART = D_MODEL // 3
        sigma = [
            u,
            4 * u0 + 2 * u2 + u1,
            4 * u1 + 2 * u0 + u2,
        ]
        ag_partners = [
            (p_x, p_y, p_z),
            (p_y, p_z, p_x),
            (p_z, p_x, p_y),
        ]

        mine = acc_ref[pl.ds(u, 1)].astype(jnp.bfloat16)
        for t in range(3):
            agbuf[t, pl.ds(sigma[t], 1)] = (
                mine[:, :, :, t * PART:(t + 1) * PART]
            )

        for r in range(3):
            width = 1 << r
            rdmas = []
            for t in range(3):
                lo = (sigma[t] // width) * width
                rdma = pltpu.make_async_remote_copy(
                    src_ref=agbuf.at[t, pl.ds(lo, width)],
                    dst_ref=agbuf.at[t, pl.ds(lo, width)],
                    send_sem=ag_send.at[t, r],
                    recv_sem=ag_recv.at[t, r],
                    device_id=(ag_partners[t][r],),
                    device_id_type=pl.DeviceIdType.MESH,
                )
                rdma.start()
                rdmas.append(rdma)
            for rdma in rdmas:
                rdma.wait()

        inv = [
            lambda s_: s_,
            lambda s_: 4 * ((s_ >> 1) & 1) + 2 * ((s_ >> 0) & 1) + ((s_ >> 2) & 1),
            lambda s_: 4 * ((s_ >> 0) & 1) + 2 * ((s_ >> 2) & 1) + ((s_ >> 1) & 1),
        ]
        for t in range(3):
            for s_ in range(N_DEV):
                j = inv[t](s_)
                out_ref[:, j * CHUNK:(j + 1) * CHUNK,
                        t * PART:(t + 1) * PART] = agbuf[t, s_]

    cshape = (B, CHUNK, D_MODEL)
    hshape = (B, CHUNK, D_MODEL // 2)
    return pl.pallas_call(
        body,
        out_shape=jax.ShapeDtypeStruct((B, SQ, D_MODEL), jnp.bfloat16),
        in_specs=[
            pl.BlockSpec(memory_space=pltpu.VMEM),
            pl.BlockSpec(memory_space=pltpu.VMEM),
            pl.BlockSpec(memory_space=pltpu.VMEM),
            pl.BlockSpec(memory_space=pltpu.VMEM),
            pl.BlockSpec(memory_space=pltpu.VMEM),
        ],
        out_specs=pl.BlockSpec(memory_space=pltpu.VMEM),
        scratch_shapes=[
            pltpu.VMEM((N_DEV,) + cshape, jnp.float32),
            pltpu.VMEM((3, N_DEV, B, CHUNK, D_MODEL // 3),
                       jnp.bfloat16),
            pltpu.VMEM((4,) + cshape, jnp.bfloat16),
            pltpu.VMEM((4,) + cshape, jnp.bfloat16),
            pltpu.VMEM((2,) + hshape, jnp.bfloat16),
            pltpu.VMEM((2,) + hshape, jnp.bfloat16),
            pltpu.VMEM((2,) + hshape, jnp.bfloat16),
            pltpu.VMEM((2,) + hshape, jnp.bfloat16),
            pltpu.VMEM((1,) + hshape, jnp.bfloat16),
            pltpu.VMEM((1,) + hshape, jnp.bfloat16),
            pltpu.VMEM((1,) + hshape, jnp.bfloat16),
            pltpu.VMEM((1,) + hshape, jnp.bfloat16),
            pltpu.SemaphoreType.DMA((B,)),
            pltpu.SemaphoreType.DMA((B,)),
            pltpu.SemaphoreType.DMA((5,)),
            pltpu.SemaphoreType.DMA((5,)),
            pltpu.SemaphoreType.DMA((3, 3)),
            pltpu.SemaphoreType.DMA((3, 3)),
        ],
        compiler_params=pltpu.CompilerParams(collective_id=0),
    )(x, Wq, K_sh, V_sh, Wo)


# device time: 40082 ns/iter; 1.2885x vs baseline; 1.2885x over previous
import os

import jax
import jax.numpy as jnp
from jax import lax
from jax.experimental import pallas as pl
from jax.experimental.pallas import tpu as pltpu

_NO_COMM = os.environ.get("KERNEL_NO_COMM") == "1"

N_DEV = 8
B = 2
SQ = 512
SKV = 512
H_LOC = 8
DH = 64
D_MODEL = 768
CHUNK = SQ // N_DEV


def kernel(x, Wq, K_ext, V_ext, Wo):
    idx = lax.axis_index("i")

    Kf = K_ext.reshape(B, SKV, 64 * DH)
    Vf = V_ext.reshape(B, SKV, 64 * DH)
    K_sh = lax.dynamic_slice_in_dim(
        Kf, idx * (H_LOC * DH), H_LOC * DH, axis=2).astype(jnp.bfloat16)
    V_sh = lax.dynamic_slice_in_dim(
        Vf, idx * (H_LOC * DH), H_LOC * DH, axis=2).astype(jnp.bfloat16)

    def body(x_ref, wq_ref, k_ref, v_ref, wo_ref, out_ref,
             acc_ref, agbuf,
             sbuf0, rbuf0, sA1, rA1, sB1, rB1, sA2, rA2, sB2, rB2,
             rs_send, rs_recv, rs2_send, rs2_recv, ag_send, ag_recv):
        my = lax.axis_index("i")
        p_z = jnp.bitwise_xor(my, 4)
        p_y = jnp.bitwise_xor(my, 3)
        p_x = jnp.bitwise_xor(my, 1)

        b2 = my // 4
        b1 = (my // 2) % 2
        b0 = my % 2
        u2 = b2
        u1 = b1
        u0 = jnp.bitwise_xor(b1, b0)
        u = 4 * u2 + 2 * u1 + u0

        barrier_sem = pltpu.get_barrier_semaphore()
        for nbr in (p_z, p_y, p_x):
            pl.semaphore_signal(
                barrier_sem, inc=1,
                device_id=(nbr,), device_id_type=pl.DeviceIdType.MESH,
            )
        pl.semaphore_wait(barrier_sem, 3)

        HALF = SQ // 2
        wq = (wq_ref[...] * 0.125).astype(jnp.bfloat16)
        wo = wo_ref[...].astype(jnp.bfloat16)
        kbf = [k_ref[b] for b in range(B)]
        vbf = [v_ref[b] for b in range(B)]

        def compute_half(r0, b):
            xb = x_ref[b, pl.ds(r0, HALF), :].astype(jnp.bfloat16)
            q = jnp.dot(xb, wq, preferred_element_type=jnp.float32)
            qbf = q.astype(jnp.bfloat16)
            qrow = lax.broadcasted_iota(jnp.int32, (HALF, SKV), 0) + r0
            qb_ = qrow // 64
            kb_ = lax.broadcasted_iota(jnp.int32, (HALF, SKV), 1) // 64
            mask = (qb_ == kb_) | (kb_ == 0) | ((qb_ + kb_) % 3 == 0)
            ctx_parts = []
            for h in range(H_LOC):
                sl = slice(h * DH, (h + 1) * DH)
                s = lax.dot_general(
                    qbf[:, sl], kbf[b][:, sl], (((1,), (1,)), ((), ())),
                    preferred_element_type=jnp.float32,
                )
                w = jnp.exp(jnp.where(mask, s, -1e9))
                rcp = 1.0 / jnp.sum(w, axis=1, keepdims=True)
                ctx_parts.append(
                    jnp.dot(w.astype(jnp.bfloat16), vbf[b][:, sl],
                            preferred_element_type=jnp.float32) * rcp
                )
            ctx = jnp.concatenate(ctx_parts, axis=1).astype(jnp.bfloat16)
            part = jnp.dot(ctx, wo, preferred_element_type=jnp.float32)
            acc_ref[pl.ds(r0 // CHUNK, HALF // CHUNK), b] = part.reshape(
                HALF // CHUNK, CHUNK, D_MODEL
            )

        send_r0 = (1 - u2) * HALF
        keep_r0 = u2 * HALF

        if _NO_COMM:
            for b in range(B):
                compute_half(send_r0, b)
                compute_half(keep_r0, b)
            for c in range(N_DEV):
                out_ref[:, c * CHUNK:(c + 1) * CHUNK, :] = (
                    acc_ref[c].astype(jnp.bfloat16))
            return

        rdma0 = []
        for b in range(B):
            compute_half(send_r0, b)
            sbuf0[:, b] = acc_ref[pl.ds(4 * (1 - u2), 4), b].astype(
                jnp.bfloat16)
            r = pltpu.make_async_remote_copy(
                src_ref=sbuf0.at[:, b],
                dst_ref=rbuf0.at[:, b],
                send_sem=rs_send.at[b],
                recv_sem=rs_recv.at[b],
                device_id=(p_z,),
                device_id_type=pl.DeviceIdType.MESH,
            )
            r.start()
            rdma0.append(r)
        for b in range(B):
            compute_half(keep_r0, b)
        for r in rdma0:
            r.wait()

        base = 4 * u2
        CA = D_MODEL // 2

        def start_rdma(src, dst, ssem, rsem, partner):
            rdma = pltpu.make_async_remote_copy(
                src_ref=src, dst_ref=dst, send_sem=ssem, recv_sem=rsem,
                device_id=(partner,), device_id_type=pl.DeviceIdType.MESH,
            )
            rdma.start()
            return rdma

        def add_r0(lo, width, c0=0, c1=D_MODEL):
            sl = pl.ds(lo, width)
            acc_ref[sl, :, :, c0:c1] = (
                acc_ref[sl, :, :, c0:c1]
                + rbuf0[pl.ds(lo - 4 * u2, width), :, :, c0:c1
                        ].astype(jnp.float32)
            )

        add_r0(base + 2 * (1 - u1), 2)
        add_r0(base + 2 * u1 + (1 - u0), 1, c0=CA)
        sA1[...] = acc_ref[pl.ds(base + 2 * (1 - u1), 2), :, :, :CA].astype(
            jnp.bfloat16)
        r_a1 = start_rdma(sA1, rA1, rs2_send.at[0], rs2_recv.at[0], p_y)
        r_b1 = []
        for k in range(2):
            sB1[k] = acc_ref[base + 2 * k + (1 - u0), :, :, CA:].astype(
                jnp.bfloat16)
            r_b1.append(start_rdma(
                sB1.at[k], rB1.at[k],
                rs2_send.at[1 + k], rs2_recv.at[1 + k], p_x))
        add_r0(base + 2 * u1 + u0, 1)
        add_r0(base + 2 * u1 + (1 - u0), 1, c1=CA)
        r_a1.wait()
        for r in r_b1:
            r.wait()

        acc_ref[pl.ds(base + 2 * u1 + (1 - u0), 1), :, :, :CA] = (
            acc_ref[pl.ds(base + 2 * u1 + (1 - u0), 1), :, :, :CA]
            + rA1[pl.ds(1 - u0, 1)].astype(jnp.float32)
        )
        acc_ref[pl.ds(base + 2 * (1 - u1) + u0, 1), :, :, CA:] = (
            acc_ref[pl.ds(base + 2 * (1 - u1) + u0, 1), :, :, CA:]
            + rB1[pl.ds(1 - u1, 1)].astype(jnp.float32)
        )

        sA2[...] = acc_ref[pl.ds(base + 2 * u1 + (1 - u0), 1), :, :, :CA
                           ].astype(jnp.bfloat16)
        r_a2 = start_rdma(sA2, rA2, rs2_send.at[3], rs2_recv.at[3], p_x)
        sB2[...] = acc_ref[pl.ds(base + 2 * (1 - u1) + u0, 1), :, :, CA:
                           ].astype(jnp.bfloat16)
        r_b2 = start_rdma(sB2, rB2, rs2_send.at[4], rs2_recv.at[4], p_y)
        acc_ref[pl.ds(u, 1), :, :, :CA] = (
            acc_ref[pl.ds(u, 1), :, :, :CA]
            + rA1[pl.ds(u0, 1)].astype(jnp.float32)
        )
        acc_ref[pl.ds(u, 1), :, :, CA:] = (
            acc_ref[pl.ds(u, 1), :, :, CA:]
            + rB1[pl.ds(u1, 1)].astype(jnp.float32)
        )
        r_a2.wait()
        r_b2.wait()
        acc_ref[pl.ds(u, 1), :, :, :CA] = (
            acc_ref[pl.ds(u, 1), :, :, :CA] + rA2[...].astype(jnp.float32)
        )
        acc_ref[pl.ds(u, 1), :, :, CA:] = (
            acc_ref[pl.ds(u, 1), :, :, CA:] + rB2[...].astype(jnp.float32)
        )

        PART = D_MODEL // 3
        sigma = [
            u,
            4 * u0 + 2 * u2 + u1,
            4 * u1 + 2 * u0 + u2,
        ]
        ag_partners = [
            (p_x, p_y, p_z),
            (p_y, p_z, p_x),
            (p_z, p_x, p_y),
        ]

        mine = acc_ref[pl.ds(u, 1)].astype(jnp.bfloat16)
        for t in range(3):
            agbuf[t, pl.ds(sigma[t], 1)] = (
                mine[:, :, :, t * PART:(t + 1) * PART]
            )

        for r in range(3):
            width = 1 << r
            rdmas = []
            for t in range(3):
                lo = (sigma[t] // width) * width
                rdma = pltpu.make_async_remote_copy(
                    src_ref=agbuf.at[t, pl.ds(lo, width)],
                    dst_ref=agbuf.at[t, pl.ds(lo, width)],
                    send_sem=ag_send.at[t, r],
                    recv_sem=ag_recv.at[t, r],
                    device_id=(ag_partners[t][r],),
                    device_id_type=pl.DeviceIdType.MESH,
                )
                rdma.start()
                rdmas.append(rdma)
            for rdma in rdmas:
                rdma.wait()

        inv = [
            lambda s_: s_,
            lambda s_: 4 * ((s_ >> 1) & 1) + 2 * ((s_ >> 0) & 1) + ((s_ >> 2) & 1),
            lambda s_: 4 * ((s_ >> 0) & 1) + 2 * ((s_ >> 2) & 1) + ((s_ >> 1) & 1),
        ]
        for t in range(3):
            for s_ in range(N_DEV):
                j = inv[t](s_)
                out_ref[:, j * CHUNK:(j + 1) * CHUNK,
                        t * PART:(t + 1) * PART] = agbuf[t, s_]

    cshape = (B, CHUNK, D_MODEL)
    hshape = (B, CHUNK, D_MODEL // 2)
    return pl.pallas_call(
        body,
        out_shape=jax.ShapeDtypeStruct((B, SQ, D_MODEL), jnp.bfloat16),
        in_specs=[
            pl.BlockSpec(memory_space=pltpu.VMEM),
            pl.BlockSpec(memory_space=pltpu.VMEM),
            pl.BlockSpec(memory_space=pltpu.VMEM),
            pl.BlockSpec(memory_space=pltpu.VMEM),
            pl.BlockSpec(memory_space=pltpu.VMEM),
        ],
        out_specs=pl.BlockSpec(memory_space=pltpu.VMEM),
        scratch_shapes=[
            pltpu.VMEM((N_DEV,) + cshape, jnp.float32),
            pltpu.VMEM((3, N_DEV, B, CHUNK, D_MODEL // 3),
                       jnp.bfloat16),
            pltpu.VMEM((4,) + cshape, jnp.bfloat16),
            pltpu.VMEM((4,) + cshape, jnp.bfloat16),
            pltpu.VMEM((2,) + hshape, jnp.bfloat16),
            pltpu.VMEM((2,) + hshape, jnp.bfloat16),
            pltpu.VMEM((2,) + hshape, jnp.bfloat16),
            pltpu.VMEM((2,) + hshape, jnp.bfloat16),
            pltpu.VMEM((1,) + hshape, jnp.bfloat16),
            pltpu.VMEM((1,) + hshape, jnp.bfloat16),
            pltpu.VMEM((1,) + hshape, jnp.bfloat16),
            pltpu.VMEM((1,) + hshape, jnp.bfloat16),
            pltpu.SemaphoreType.DMA((B,)),
            pltpu.SemaphoreType.DMA((B,)),
            pltpu.SemaphoreType.DMA((5,)),
            pltpu.SemaphoreType.DMA((5,)),
            pltpu.SemaphoreType.DMA((3, 3)),
            pltpu.SemaphoreType.DMA((3, 3)),
        ],
        compiler_params=pltpu.CompilerParams(collective_id=0),
    )(x, Wq, K_sh, V_sh, Wo)
